# baseline (speedup 1.0000x reference)
import jax
import jax.numpy as jnp
from jax import lax
from jax.experimental import pallas as pl
from jax.experimental.pallas import tpu as pltpu
from jax.experimental.pallas import tpu_sc as plsc

_L = 16
_NC = 2
_NS = 16
_NW = _NC * _NS
_B = 16384
_BW = _B // _NW
_T = 1001


def _gather_body(t_hbm, betas_hbm, out_hbm, table_v, idx_v, out_v,
                 sem_t, sem_i):
    wid = lax.axis_index("s") * _NC + lax.axis_index("c")
    base = wid * _BW
    cp_tab = pltpu.async_copy(betas_hbm, table_v, sem_t)
    cp_idx = pltpu.async_copy(t_hbm.at[pl.ds(base, _BW)], idx_v, sem_i)
    cp_tab.wait()
    cp_idx.wait()

    def _step(j, carry):
        idx = idx_v[pl.ds(j * _L, _L)]
        out_v[pl.ds(j * _L, _L)] = plsc.load_gather(table_v, [idx])
        return carry
    lax.fori_loop(0, _BW // _L, _step, 0)
    pltpu.sync_copy(out_v, out_hbm.at[pl.ds(base, _BW)])


def kernel(t_int, betas):
    mesh = plsc.VectorSubcoreMesh(
        core_axis_name="c", subcore_axis_name="s",
        num_cores=_NC, num_subcores=_NS)
    return pl.kernel(
        _gather_body,
        out_type=jax.ShapeDtypeStruct((_B,), jnp.float32),
        mesh=mesh,
        compiler_params=pltpu.CompilerParams(
            needs_layout_passes=False,
            disable_bounds_checks=True,
            disable_semaphore_checks=True,
            skip_device_barrier=True,
        ),
        scratch_types=[
            pltpu.VMEM((_T,), jnp.float32),
            pltpu.VMEM((_BW,), jnp.int32),
            pltpu.VMEM((_BW,), jnp.float32),
            pltpu.SemaphoreType.DMA,
            pltpu.SemaphoreType.DMA,
        ],
    )(t_int, betas)


# speedup vs baseline: 1.0381x; 1.0381x over previous
import jax
import jax.numpy as jnp
from jax import lax
from jax.experimental import pallas as pl
from jax.experimental.pallas import tpu as pltpu
from jax.experimental.pallas import tpu_sc as plsc

_L = 16
_NC = 2
_NS = 16
_NW = _NC * _NS
_B = 16384
_BW = _B // _NW
_T = 1001


def _gather_body(t_hbm, betas_hbm, out_hbm, table_s, idx_v, out_v,
                 sem_t, sem_i):
    sid = lax.axis_index("s")
    wid = sid * _NC + lax.axis_index("c")
    base = wid * _BW
    cp_idx = pltpu.async_copy(t_hbm.at[pl.ds(base, _BW)], idx_v, sem_i)
    @pl.when(sid == 0)
    def _():
        pltpu.sync_copy(betas_hbm, table_s)
    plsc.subcore_barrier()
    cp_idx.wait()
    pltpu.async_copy(table_s.at[idx_v], out_v, sem_t).wait()
    pltpu.sync_copy(out_v, out_hbm.at[pl.ds(base, _BW)])


def kernel(t_int, betas):
    mesh = plsc.VectorSubcoreMesh(
        core_axis_name="c", subcore_axis_name="s",
        num_cores=_NC, num_subcores=_NS)
    return pl.kernel(
        _gather_body,
        out_type=jax.ShapeDtypeStruct((_B,), jnp.float32),
        mesh=mesh,
        compiler_params=pltpu.CompilerParams(
            needs_layout_passes=False,
            disable_bounds_checks=True,
            disable_semaphore_checks=True,
            skip_device_barrier=True,
        ),
        scratch_types=[
            pltpu.VMEM_SHARED((_T,), jnp.float32),
            pltpu.VMEM((_BW,), jnp.int32),
            pltpu.VMEM((_BW,), jnp.float32),
            pltpu.SemaphoreType.DMA,
            pltpu.SemaphoreType.DMA,
        ],
    )(t_int, betas)
